# baseline (speedup 1.0000x reference)
import functools
import math

import jax
import jax.numpy as jnp
from jax import lax
from jax.experimental import pallas as pl
from jax.experimental.pallas import tpu as pltpu
from jax.experimental.pallas import tpu_sc as plsc

BJ_A1 = 0.4
BJ_A2 = 4.0

NC = 2
NS = 16
LANES = 128
CH = 1024
K = CH // LANES


def _prep_body(nelem, nz, n, npk, na_ref, an_ref, gp_ref, pk_ref, tabs_ref):
    m = na_ref[0, :]
    e = jnp.zeros(m.shape, jnp.int32)
    for j in range(1, nelem):
        vj = na_ref[j, :]
        gt = vj > m
        e = jnp.where(gt, j, e)
        m = jnp.where(gt, vj, m)
    groups = math.ceil(n / npk)
    ep = jnp.concatenate([e, jnp.zeros((groups * npk - n,), jnp.int32)])
    pk = jnp.zeros((npk,), jnp.int32)
    for g in range(groups):
        pk = pk + ((lax.slice(ep, (g * npk,), ((g + 1) * npk,)) & 15) << (4 * g))
    pk_ref[0, :] = pk

    zz = an_ref[0, :]
    onehot = (zz[:, None] == lax.broadcasted_iota(jnp.int32, (nelem, nz), 1)
              ).astype(jnp.float32)
    p10 = jnp.dot(onehot, gp_ref[...], preferred_element_type=jnp.float32)
    c6 = p10[:, 3]
    rr = p10[:, 2]
    c6ij = jnp.sqrt(c6[:, None] * c6[None, :])
    rij = jnp.sqrt(rr[:, None] * rr[None, :])
    r0 = BJ_A1 * jnp.sqrt(rij) + BJ_A2
    r02 = r0 * r0
    r04 = r02 * r02
    r08 = r04 * r04
    tab_b = 6.0 * (r08 * r04 * r02)
    rij2 = rij * rij
    tab_d = rij2 * rij2 * rij2
    tab_a = -0.5 * c6ij
    tabs_ref[...] = jnp.stack([tab_a, tab_b, tab_d])


def _sc_body(nelem, chunks, per_w, npk_shift, e_off, ei_ref, x_ref, pk_ref,
             tabs_ref, p_ref, pk_v, ta_v, tb_v, td_v,
             snd0, rcv0, x0, snd1, rcv1, x1, snd2, rcv2, x2, snd3, rcv3, x3, acc_v,
             sem_in0, sem_in1, sem_in2, sem_in3):
    c = lax.axis_index("c")
    s = lax.axis_index("s")
    w = s * NC + c
    nw = NC * NS
    n_pad = acc_v.shape[0]
    npk_mask = (1 << npk_shift) - 1
    bufs = ((snd0, rcv0, x0, sem_in0), (snd1, rcv1, x1, sem_in1),
            (snd2, rcv2, x2, sem_in2), (snd3, rcv3, x3, sem_in3))

    def start_inputs(cid, p):
        snd_v, rcv_v, x_v, sem_in = bufs[p]
        off = cid * CH
        pltpu.async_copy(ei_ref.at[pl.ds(off, CH)], snd_v, sem_in)
        pltpu.async_copy(ei_ref.at[pl.ds(e_off + off, CH)], rcv_v, sem_in)
        pltpu.async_copy(x_ref.at[pl.ds(off, CH)], x_v, sem_in)

    @pl.when(w < chunks)
    def _prime0():
        start_inputs(w, 0)

    @pl.when(w + nw < chunks)
    def _prime1():
        start_inputs(w + nw, 1)

    @pl.when(w + 2 * nw < chunks)
    def _prime2():
        start_inputs(w + 2 * nw, 2)

    pltpu.sync_copy(pk_ref, pk_v)
    pltpu.sync_copy(tabs_ref.at[pl.ds(0, LANES)], ta_v)
    pltpu.sync_copy(tabs_ref.at[pl.ds(LANES, LANES)], tb_v)
    pltpu.sync_copy(tabs_ref.at[pl.ds(2 * LANES, LANES)], td_v)

    zero16 = jnp.zeros((16,), jnp.float32)

    @plsc.parallel_loop(0, n_pad, 16, unroll=8)
    def _zero(i):
        acc_v[pl.ds(i, 16)] = zero16

    def wait_inputs(cid, p):
        snd_v, rcv_v, x_v, sem_in = bufs[p]
        off = cid * CH
        pltpu.make_async_copy(ei_ref.at[pl.ds(off, CH)], snd_v, sem_in).wait()
        pltpu.make_async_copy(ei_ref.at[pl.ds(e_off + off, CH)], rcv_v,
                              sem_in).wait()
        pltpu.make_async_copy(x_ref.at[pl.ds(off, CH)], x_v, sem_in).wait()

    def unpack_elem(idx):
        word = plsc.load_gather(pk_v, [idx & npk_mask])
        sh = (lax.shift_right_logical(idx, npk_shift - 2)) & 0x7C
        return lax.shift_right_logical(word, sh) & 15

    def compute(p):
        snd_v, rcv_v, x_v, _ = bufs[p]

        @plsc.parallel_loop(0, CH, 16, unroll=8)
        def _body(i):
            su = snd_v[pl.ds(i, 16)]
            ru = rcv_v[pl.ds(i, 16)]
            eu = unpack_elem(su)
            ev = unpack_elem(ru)
            pair = eu * nelem + ev
            a = plsc.load_gather(ta_v, [pair])
            b = plsc.load_gather(tb_v, [pair])
            d = plsc.load_gather(td_v, [pair])
            r = x_v[pl.ds(i, 16)]
            r2 = r * r
            r4 = r2 * r2
            r8 = r4 * r4
            r14 = r8 * r4 * r2
            r6 = r4 * r2
            den = (r14 + b) * (d + r6)
            q = (a * r14) / jnp.maximum(den, 1e-30)
            plsc.addupdate_scatter(acc_v, [ru], q)

    def stage(i3, p):
        idx = 4 * i3 + p
        cid = w + nw * idx

        @pl.when(cid < chunks)
        def _do():
            wait_inputs(cid, p)

            @pl.when(cid + 3 * nw < chunks)
            def _prefetch():
                start_inputs(cid + 3 * nw, (p + 3) % 4)

            compute(p)

    def chunk_body(i3, carry):
        stage(i3, 0)
        stage(i3, 1)
        stage(i3, 2)
        stage(i3, 3)
        return carry

    lax.fori_loop(0, (per_w + 3) // 4, chunk_body, 0)

    pltpu.sync_copy(acc_v, p_ref.at[pl.ds(w * n_pad, n_pad)])


def _add_body(p_ref, v_ref):
    v_ref[0, :] = jnp.sum(p_ref[...], axis=0)


def kernel(x, node_attrs, edge_index, atomic_numbers, gnb_params):
    n = node_attrs.shape[0]
    nelem = node_attrs.shape[1]
    e = x.shape[0]
    nz = gnb_params.shape[0]
    npk_shift = max(4, math.ceil(math.log2(n / 8)))
    npk = 1 << npk_shift

    na_t = node_attrs.T
    pk2d, tabs3 = pl.pallas_call(
        functools.partial(_prep_body, nelem, nz, n, npk),
        out_shape=[
            jax.ShapeDtypeStruct((1, npk), jnp.int32),
            jax.ShapeDtypeStruct((3, nelem, nelem), jnp.float32),
        ],
    )(na_t, atomic_numbers.reshape(1, nelem), gnb_params)
    pk = pk2d.reshape(npk)
    tabs = jnp.pad(tabs3.reshape(3, nelem * nelem),
                   ((0, 0), (0, LANES - nelem * nelem))).reshape(3 * LANES)

    e_pad = math.ceil(e / CH) * CH
    ei = edge_index
    xf = x.reshape(e)
    if e_pad != e:
        ei = jnp.pad(ei, ((0, 0), (0, e_pad - e)))
        xf = jnp.pad(xf, (0, e_pad - e))
    chunks = e_pad // CH
    nw = NC * NS
    per_w = math.ceil(chunks / nw)
    ei1 = ei.reshape(2 * e_pad)
    n_pad = math.ceil(n / (NS * 8)) * (NS * 8)

    mesh = plsc.VectorSubcoreMesh(core_axis_name="c", subcore_axis_name="s",
                                  num_cores=NC, num_subcores=NS)
    partials = pl.kernel(
        functools.partial(_sc_body, nelem, chunks, per_w, npk_shift, e_pad),
        out_type=jax.ShapeDtypeStruct((nw * n_pad,), jnp.float32),
        mesh=mesh,
        compiler_params=pltpu.CompilerParams(needs_layout_passes=False),
        scratch_types=(
            [pltpu.VMEM((npk,), jnp.int32)]
            + [pltpu.VMEM((LANES,), jnp.float32)] * 3
            + [pltpu.VMEM((CH,), jnp.int32),
               pltpu.VMEM((CH,), jnp.int32),
               pltpu.VMEM((CH,), jnp.float32)] * 4
            + [pltpu.VMEM((n_pad,), jnp.float32)]
            + [pltpu.SemaphoreType.DMA] * 4
        ),
    )(ei1, xf, pk, tabs)

    v2d = pl.pallas_call(
        _add_body,
        out_shape=jax.ShapeDtypeStruct((1, n_pad), jnp.float32),
    )(partials.reshape(nw, n_pad))
    return v2d.reshape(n_pad)[:n].astype(x.dtype)


# speedup vs baseline: 1.0058x; 1.0058x over previous
import functools
import math

import jax
import jax.numpy as jnp
from jax import lax
from jax.experimental import pallas as pl
from jax.experimental.pallas import tpu as pltpu
from jax.experimental.pallas import tpu_sc as plsc

BJ_A1 = 0.4
BJ_A2 = 4.0

NC = 2
NS = 16
LANES = 128
CH = 1024
K = CH // LANES


def _prep_body(nelem, nz, n, npk, na_ref, an_ref, gp_ref, pk_ref, tabs_ref):
    m = na_ref[0, :]
    e = jnp.zeros(m.shape, jnp.int32)
    for j in range(1, nelem):
        vj = na_ref[j, :]
        gt = vj > m
        e = jnp.where(gt, j, e)
        m = jnp.where(gt, vj, m)
    groups = math.ceil(n / npk)
    ep = jnp.concatenate([e, jnp.zeros((groups * npk - n,), jnp.int32)])
    pk = jnp.zeros((npk,), jnp.int32)
    for g in range(groups):
        pk = pk + ((lax.slice(ep, (g * npk,), ((g + 1) * npk,)) & 15) << (4 * g))
    pk_ref[0, :] = pk

    zz = an_ref[0, :]
    onehot = (zz[:, None] == lax.broadcasted_iota(jnp.int32, (nelem, nz), 1)
              ).astype(jnp.float32)
    p10 = jnp.dot(onehot, gp_ref[...], preferred_element_type=jnp.float32)
    c6 = p10[:, 3]
    rr = p10[:, 2]
    c6ij = jnp.sqrt(c6[:, None] * c6[None, :])
    rij = jnp.sqrt(rr[:, None] * rr[None, :])
    r0 = BJ_A1 * jnp.sqrt(rij) + BJ_A2
    r02 = r0 * r0
    r04 = r02 * r02
    r08 = r04 * r04
    tab_b = 6.0 * (r08 * r04 * r02)
    rij2 = rij * rij
    tab_d = rij2 * rij2 * rij2
    tab_a = -0.5 * c6ij
    tabs_ref[...] = jnp.stack([tab_a, tab_b, tab_d])


def _sc_body(nelem, chunks, per_w, npk_shift, e_off, ei_ref, x_ref, pk_ref,
             tabs_ref, p_ref, pk_v, ta_v, tb_v, td_v,
             snd0, rcv0, x0, snd1, rcv1, x1, snd2, rcv2, x2, acc_v,
             sem_in0, sem_in1, sem_in2):
    c = lax.axis_index("c")
    s = lax.axis_index("s")
    w = s * NC + c
    nw = NC * NS
    n_pad = acc_v.shape[0]
    npk_mask = (1 << npk_shift) - 1
    bufs = ((snd0, rcv0, x0, sem_in0), (snd1, rcv1, x1, sem_in1),
            (snd2, rcv2, x2, sem_in2))

    def start_inputs(cid, p):
        snd_v, rcv_v, x_v, sem_in = bufs[p]
        off = cid * CH
        pltpu.async_copy(ei_ref.at[pl.ds(off, CH)], snd_v, sem_in)
        pltpu.async_copy(ei_ref.at[pl.ds(e_off + off, CH)], rcv_v, sem_in)
        pltpu.async_copy(x_ref.at[pl.ds(off, CH)], x_v, sem_in)

    @pl.when(w < chunks)
    def _prime0():
        start_inputs(w, 0)

    @pl.when(w + nw < chunks)
    def _prime1():
        start_inputs(w + nw, 1)

    pltpu.sync_copy(pk_ref, pk_v)
    pltpu.sync_copy(tabs_ref.at[pl.ds(0, LANES)], ta_v)
    pltpu.sync_copy(tabs_ref.at[pl.ds(LANES, LANES)], tb_v)
    pltpu.sync_copy(tabs_ref.at[pl.ds(2 * LANES, LANES)], td_v)

    zero16 = jnp.zeros((16,), jnp.float32)

    @plsc.parallel_loop(0, n_pad, 16, unroll=8)
    def _zero(i):
        acc_v[pl.ds(i, 16)] = zero16

    def wait_inputs(cid, p):
        snd_v, rcv_v, x_v, sem_in = bufs[p]
        off = cid * CH
        pltpu.make_async_copy(ei_ref.at[pl.ds(off, CH)], snd_v, sem_in).wait()
        pltpu.make_async_copy(ei_ref.at[pl.ds(e_off + off, CH)], rcv_v,
                              sem_in).wait()
        pltpu.make_async_copy(x_ref.at[pl.ds(off, CH)], x_v, sem_in).wait()

    def unpack_elem(idx):
        word = plsc.load_gather(pk_v, [idx & npk_mask])
        sh = (lax.shift_right_logical(idx, npk_shift - 2)) & 0x7C
        return lax.shift_right_logical(word, sh) & 15

    def compute(p):
        snd_v, rcv_v, x_v, _ = bufs[p]

        @plsc.parallel_loop(0, CH, 16, unroll=8)
        def _body(i):
            su = snd_v[pl.ds(i, 16)]
            ru = rcv_v[pl.ds(i, 16)]
            eu = unpack_elem(su)
            ev = unpack_elem(ru)
            pair = eu * nelem + ev
            a = plsc.load_gather(ta_v, [pair])
            b = plsc.load_gather(tb_v, [pair])
            d = plsc.load_gather(td_v, [pair])
            r = x_v[pl.ds(i, 16)]
            r2 = r * r
            r4 = r2 * r2
            r8 = r4 * r4
            r14 = r8 * r4 * r2
            r6 = r4 * r2
            den = (r14 + b) * (d + r6)
            q = (a * r14) / jnp.maximum(den, 1e-30)
            plsc.addupdate_scatter(acc_v, [ru], q)

    def stage(i3, p):
        idx = 3 * i3 + p
        cid = w + nw * idx

        @pl.when(cid < chunks)
        def _do():
            wait_inputs(cid, p)

            @pl.when(cid + 2 * nw < chunks)
            def _prefetch():
                start_inputs(cid + 2 * nw, (p + 2) % 3)

            compute(p)

    def chunk_body(i3, carry):
        stage(i3, 0)
        stage(i3, 1)
        stage(i3, 2)
        return carry

    lax.fori_loop(0, (per_w + 2) // 3, chunk_body, 0)

    pltpu.sync_copy(acc_v, p_ref.at[pl.ds(w * n_pad, n_pad)])


def _add_body(p_ref, v_ref):
    v_ref[0, :] = jnp.sum(p_ref[...], axis=0)


def kernel(x, node_attrs, edge_index, atomic_numbers, gnb_params):
    n = node_attrs.shape[0]
    nelem = node_attrs.shape[1]
    e = x.shape[0]
    nz = gnb_params.shape[0]
    npk_shift = max(4, math.ceil(math.log2(n / 8)))
    npk = 1 << npk_shift

    na_t = node_attrs.T
    pk2d, tabs3 = pl.pallas_call(
        functools.partial(_prep_body, nelem, nz, n, npk),
        out_shape=[
            jax.ShapeDtypeStruct((1, npk), jnp.int32),
            jax.ShapeDtypeStruct((3, nelem, nelem), jnp.float32),
        ],
    )(na_t, atomic_numbers.reshape(1, nelem), gnb_params)
    pk = pk2d.reshape(npk)
    tabs = jnp.pad(tabs3.reshape(3, nelem * nelem),
                   ((0, 0), (0, LANES - nelem * nelem))).reshape(3 * LANES)

    e_pad = math.ceil(e / CH) * CH
    ei = edge_index
    xf = x.reshape(e)
    if e_pad != e:
        ei = jnp.pad(ei, ((0, 0), (0, e_pad - e)))
        xf = jnp.pad(xf, (0, e_pad - e))
    chunks = e_pad // CH
    nw = NC * NS
    per_w = math.ceil(chunks / nw)
    ei1 = ei.reshape(2 * e_pad)
    n_pad = math.ceil(n / (NS * 8)) * (NS * 8)

    mesh = plsc.VectorSubcoreMesh(core_axis_name="c", subcore_axis_name="s",
                                  num_cores=NC, num_subcores=NS)
    partials = pl.kernel(
        functools.partial(_sc_body, nelem, chunks, per_w, npk_shift, e_pad),
        out_type=jax.ShapeDtypeStruct((nw * n_pad,), jnp.float32),
        mesh=mesh,
        compiler_params=pltpu.CompilerParams(needs_layout_passes=False),
        scratch_types=(
            [pltpu.VMEM((npk,), jnp.int32)]
            + [pltpu.VMEM((LANES,), jnp.float32)] * 3
            + [pltpu.VMEM((CH,), jnp.int32),
               pltpu.VMEM((CH,), jnp.int32),
               pltpu.VMEM((CH,), jnp.float32)] * 3
            + [pltpu.VMEM((n_pad,), jnp.float32)]
            + [pltpu.SemaphoreType.DMA] * 3
        ),
    )(ei1, xf, pk, tabs)

    v2d = pl.pallas_call(
        _add_body,
        out_shape=jax.ShapeDtypeStruct((1, n_pad), jnp.float32),
    )(partials.reshape(nw, n_pad))
    return v2d.reshape(n_pad)[:n].astype(x.dtype)
